# baseline (speedup 1.0000x reference)
import jax
import jax.numpy as jnp
from jax import lax
from jax.experimental import pallas as pl
from jax.experimental.pallas import tpu as pltpu
from jax.experimental.pallas import tpu_sc as plsc

_HIDDEN = 1024
_BUF = 4096
_NC = 2
_NS = 16
_ROW = _BUF // (_NC * _NS)
_ROW_BLOCK = 4096


def _sc_segment_sum_body(idx_hbm, sm_hbm, w_hbm, idx_v, sm_v, zero_v, w_shared,
                         sem_i, sem_s):
    cid = lax.axis_index("c")
    sid = lax.axis_index("s")
    row = sid * _NC + cid

    cp_i = pltpu.async_copy(idx_hbm.at[pl.ds(row * _ROW, _ROW)], idx_v, sem_i)
    cp_s = pltpu.async_copy(sm_hbm.at[pl.ds(row * _ROW, _ROW)], sm_v, sem_s)

    zseg = _HIDDEN // _NS
    zeros = jnp.zeros((16,), jnp.float32)
    for k in range(zseg // 16):
        zero_v[pl.ds(k * 16, 16)] = zeros
    pltpu.sync_copy(zero_v, w_shared.at[pl.ds(sid * zseg, zseg)])

    cp_i.wait()
    cp_s.wait()
    plsc.subcore_barrier()
    pltpu.sync_copy(sm_v, w_shared.at[idx_v], add=True)
    plsc.subcore_barrier()

    @pl.when(sid == 0)
    def _():
        pltpu.sync_copy(w_shared, w_hbm.at[cid])


def _sc_segment_sum(top_k_indices, smooth_f32):
    mesh = plsc.VectorSubcoreMesh(core_axis_name="c", subcore_axis_name="s")
    fn = pl.kernel(
        _sc_segment_sum_body,
        out_type=jax.ShapeDtypeStruct((_NC, _HIDDEN), jnp.float32),
        mesh=mesh,
        scratch_types=[
            pltpu.VMEM((_ROW,), jnp.int32),
            pltpu.VMEM((_ROW,), jnp.float32),
            pltpu.VMEM((_HIDDEN // _NS,), jnp.float32),
            pltpu.VMEM_SHARED((_HIDDEN,), jnp.float32),
            pltpu.SemaphoreType.DMA,
            pltpu.SemaphoreType.DMA,
        ],
    )
    return fn(top_k_indices, smooth_f32)


def _tc_scale_body(wp_ref, x_ref, o_ref):
    w = jnp.sum(wp_ref[...], axis=0, keepdims=True)
    o_ref[...] = x_ref[...] * w.astype(jnp.bfloat16)


_COL_BLOCK = 512


def _tc_scale(x, w_pair):
    tokens = x.shape[0]
    grid = (tokens // _ROW_BLOCK, _HIDDEN // _COL_BLOCK)
    return pl.pallas_call(
        _tc_scale_body,
        grid=grid,
        in_specs=[
            pl.BlockSpec((_NC, _COL_BLOCK), lambda i, j: (0, j)),
            pl.BlockSpec((_ROW_BLOCK, _COL_BLOCK), lambda i, j: (i, j)),
        ],
        out_specs=pl.BlockSpec((_ROW_BLOCK, _COL_BLOCK), lambda i, j: (i, j)),
        out_shape=jax.ShapeDtypeStruct((tokens, _HIDDEN), jnp.bfloat16),
    )(w_pair, x)


@jax.jit
def kernel(x, smooth, top_k_indices):
    w_pair = _sc_segment_sum(top_k_indices, smooth.astype(jnp.float32))
    return _tc_scale(x, w_pair)


# speedup vs baseline: 1.0343x; 1.0343x over previous
import jax
import jax.numpy as jnp
from jax import lax
from jax.experimental import pallas as pl
from jax.experimental.pallas import tpu as pltpu
from jax.experimental.pallas import tpu_sc as plsc

_HIDDEN = 1024
_BUF = 4096
_NC = 2
_NS = 16
_ROW = _BUF // (_NC * _NS)
_ROW_BLOCK = 4096


def _sc_segment_sum_body(idx_hbm, sm_hbm, w_hbm, idx_v, sm_v, zero_v, w_shared,
                         sem_i, sem_s):
    cid = lax.axis_index("c")
    sid = lax.axis_index("s")
    row = sid * _NC + cid

    cp_i = pltpu.async_copy(idx_hbm.at[pl.ds(row * _ROW, _ROW)], idx_v, sem_i)
    cp_s = pltpu.async_copy(sm_hbm.at[pl.ds(row * _ROW, _ROW)], sm_v, sem_s)

    zseg = _HIDDEN // _NS
    zeros = jnp.zeros((16,), jnp.float32)
    for k in range(zseg // 16):
        zero_v[pl.ds(k * 16, 16)] = zeros
    pltpu.sync_copy(zero_v, w_shared.at[pl.ds(sid * zseg, zseg)])

    cp_i.wait()
    cp_s.wait()
    plsc.subcore_barrier()
    pltpu.sync_copy(sm_v, w_shared.at[idx_v], add=True)
    plsc.subcore_barrier()

    @pl.when(sid == 0)
    def _():
        pltpu.sync_copy(w_shared, w_hbm.at[cid])


def _sc_segment_sum(top_k_indices, smooth_f32):
    mesh = plsc.VectorSubcoreMesh(core_axis_name="c", subcore_axis_name="s")
    fn = pl.kernel(
        _sc_segment_sum_body,
        out_type=jax.ShapeDtypeStruct((_NC, _HIDDEN), jnp.float32),
        mesh=mesh,
        scratch_types=[
            pltpu.VMEM((_ROW,), jnp.int32),
            pltpu.VMEM((_ROW,), jnp.float32),
            pltpu.VMEM((_HIDDEN // _NS,), jnp.float32),
            pltpu.VMEM_SHARED((_HIDDEN,), jnp.float32),
            pltpu.SemaphoreType.DMA,
            pltpu.SemaphoreType.DMA,
        ],
    )
    return fn(top_k_indices, smooth_f32)


def _tc_scale_body(wp_ref, x_ref, o_ref):
    w = jnp.sum(wp_ref[...], axis=0, keepdims=True)
    o_ref[...] = x_ref[...] * w.astype(jnp.bfloat16)


def _tc_scale(x, w_pair):
    tokens = x.shape[0]
    grid = (tokens // _ROW_BLOCK,)
    return pl.pallas_call(
        _tc_scale_body,
        grid=grid,
        in_specs=[
            pl.BlockSpec((_NC, _HIDDEN), lambda i: (0, 0)),
            pl.BlockSpec((_ROW_BLOCK, _HIDDEN), lambda i: (i, 0)),
        ],
        out_specs=pl.BlockSpec((_ROW_BLOCK, _HIDDEN), lambda i: (i, 0)),
        out_shape=jax.ShapeDtypeStruct((tokens, _HIDDEN), jnp.bfloat16),
    )(w_pair, x)


@jax.jit
def kernel(x, smooth, top_k_indices):
    w_pair = _sc_segment_sum(top_k_indices, smooth.astype(jnp.float32))
    return _tc_scale(x, w_pair)
